# Initial kernel scaffold; baseline (speedup 1.0000x reference)
import jax
import jax.numpy as jnp
from jax.experimental import pallas as pl


def kernel(x, edge_index, edge_attr, W_pre, b_pre, Wl, Wr, b_conv, ln_w, ln_b):
    raise NotImplementedError("write your pallas kernel here")


# speedup vs baseline: 2.9654x; 2.9654x over previous
import functools

import jax
import jax.numpy as jnp
import numpy as np
from jax import lax
from jax.experimental import pallas as pl
from jax.experimental.pallas import tpu as pltpu
from jax.experimental.pallas import tpu_sc as plsc

N = 10000
H = 128
E = 320000
L = 3
XPAD = 256

NW = 32
CHUNK = 128
CH = 80
HCH = CH // 2
EPAD = NW * CH * CHUNK
CHA = 160
SGA = 40
NSA = 4
NPAD = 10112
RPT = NPAD // 16

BN = 2000
NB = N // BN

_MESH = plsc.VectorSubcoreMesh(core_axis_name="c", subcore_axis_name="s")


@functools.partial(
    pl.kernel,
    out_type=jax.ShapeDtypeStruct((NPAD, H), jnp.float32),
    mesh=_MESH,
    scratch_types=[
        pltpu.VMEM((SGA, CHUNK), jnp.int32),
        pltpu.VMEM((SGA, CHUNK), jnp.int32),
        pltpu.VMEM((CHUNK, H), jnp.float32),
        pltpu.VMEM((CHUNK, H), jnp.float32),
        pltpu.VMEM_SHARED((NPAD, H), jnp.float32),
        pltpu.SemaphoreType.DMA,
    ],
)
def _sc_segsum(hid_hbm, src_hbm, dst_hbm,
               out_hbm, src_v, dst_v, rows0_v, rows1_v, agg_s, sem):
    cid = lax.axis_index("c")
    sid = lax.axis_index("s")
    base = sid * RPT

    @pl.when(cid == 0)
    def _core_values():
        def zrow(r, carry):
            for k in range(H // 16):
                rows0_v[r, pl.ds(k * 16, 16)] = jnp.zeros((16,), jnp.float32)
            return carry

        lax.fori_loop(0, CHUNK, zrow, 0)
        for blk in range(RPT // CHUNK):
            pltpu.sync_copy(rows0_v, agg_s.at[pl.ds(base + blk * CHUNK, CHUNK)])
        pltpu.sync_copy(rows0_v.at[pl.ds(0, RPT % CHUNK)],
                        agg_s.at[pl.ds(base + (RPT // CHUNK) * CHUNK, RPT % CHUNK)])
        plsc.subcore_barrier()
        for hlf in range(NSA):
            pltpu.sync_copy(src_hbm.at[sid, pl.ds(hlf * SGA, SGA)],
                            src_v.at[pl.ds(0, SGA)])
            pltpu.sync_copy(dst_hbm.at[sid, pl.ds(hlf * SGA, SGA)],
                            dst_v.at[pl.ds(0, SGA)])
            pltpu.async_copy(hid_hbm.at[src_v.at[0]], rows0_v, sem)

            def body(j2, carry):
                j = 2 * j2
                pltpu.async_copy(hid_hbm.at[src_v.at[j + 1]], rows1_v, sem)
                pltpu.make_async_copy(hid_hbm.at[src_v.at[j]], rows0_v, sem).wait()
                pltpu.sync_copy(rows0_v, agg_s.at[dst_v.at[j]], add=True)
                nxt = jnp.minimum(j + 2, SGA - 1)
                pltpu.async_copy(hid_hbm.at[src_v.at[nxt]], rows0_v, sem)
                pltpu.make_async_copy(hid_hbm.at[src_v.at[j + 1]], rows1_v, sem).wait()
                pltpu.sync_copy(rows1_v, agg_s.at[dst_v.at[j + 1]], add=True)
                return carry

            lax.fori_loop(0, SGA // 2, body, 0)
            pltpu.make_async_copy(hid_hbm.at[src_v.at[SGA - 1]], rows0_v, sem).wait()
        plsc.subcore_barrier()
        pltpu.sync_copy(agg_s.at[pl.ds(base, RPT)], out_hbm.at[pl.ds(base, RPT)])


@functools.partial(
    pl.kernel,
    out_type=jax.ShapeDtypeStruct((2, NPAD, H), jnp.float32),
    mesh=_MESH,
    scratch_types=[
        pltpu.VMEM((CH, CHUNK), jnp.int32),
        pltpu.VMEM((CHUNK, H), jnp.float32),
        pltpu.VMEM_SHARED((NPAD, H), jnp.float32),
    ],
)
def _sc_degree(dst_hbm, out_hbm, dst_v, ones_v, deg_s):
    cid = lax.axis_index("c")
    sid = lax.axis_index("s")
    wid = sid * 2 + cid
    base = sid * RPT

    def fill(r, carry):
        for k in range(H // 16):
            ones_v[r, pl.ds(k * 16, 16)] = jnp.zeros((16,), jnp.float32)
        return carry

    lax.fori_loop(0, CHUNK, fill, 0)
    for blk in range(RPT // CHUNK):
        pltpu.sync_copy(ones_v, deg_s.at[pl.ds(base + blk * CHUNK, CHUNK)])
    pltpu.sync_copy(ones_v.at[pl.ds(0, RPT % CHUNK)],
                    deg_s.at[pl.ds(base + (RPT // CHUNK) * CHUNK, RPT % CHUNK)])

    def fill1(r, carry):
        for k in range(H // 16):
            ones_v[r, pl.ds(k * 16, 16)] = jnp.ones((16,), jnp.float32)
        return carry

    lax.fori_loop(0, CHUNK, fill1, 0)
    pltpu.sync_copy(dst_hbm.at[wid], dst_v)
    plsc.subcore_barrier()

    def body(j, carry):
        pltpu.sync_copy(ones_v, deg_s.at[dst_v.at[j]], add=True)
        return carry

    lax.fori_loop(0, CH, body, 0)
    plsc.subcore_barrier()
    pltpu.sync_copy(deg_s.at[pl.ds(base, RPT)], out_hbm.at[cid, pl.ds(base, RPT)])


def _dot(a, b):
    return lax.dot_general(a, b, (((1,), (0,)), ((), ())),
                           preferred_element_type=jnp.float32)


def _tc_pre_body(x_ref, w_ref, b_ref, o_ref):
    o_ref[...] = jnp.maximum(_dot(x_ref[...], w_ref[...]) + b_ref[...], 0.0)


def _tc_pre(x, wpt, b):
    xd = x.shape[1]
    return pl.pallas_call(
        _tc_pre_body,
        grid=(NB,),
        in_specs=[
            pl.BlockSpec((BN, xd), lambda i: (i, 0)),
            pl.BlockSpec((xd, H), lambda i: (0, 0)),
            pl.BlockSpec((1, H), lambda i: (0, 0)),
        ],
        out_specs=pl.BlockSpec((BN, H), lambda i: (i, 0)),
        out_shape=jax.ShapeDtypeStruct((N, H), jnp.float32),
    )(x, wpt, b)


def _tc_layer_body(p, d0, d1, h, wl, wr, b, o_ref, s_ref, q_ref):
    i = pl.program_id(0)
    cnt = d0[:, 0:1] + d1[:, 0:1]
    inv = 1.0 / jnp.maximum(cnt, 1.0)
    agg = p[...] * inv
    o = jnp.maximum(_dot(agg, wl[...]) + _dot(h[...], wr[...]) + b[...], 0.0)
    o_ref[...] = o

    @pl.when(i == 0)
    def _init():
        s_ref[...] = jnp.zeros_like(s_ref)
        q_ref[...] = jnp.zeros_like(q_ref)

    s_ref[...] += jnp.sum(o)
    q_ref[...] += jnp.sum(o * o)


def _tc_layer(p, d0, d1, h, wlt, wrt, b):
    return pl.pallas_call(
        _tc_layer_body,
        grid=(NB,),
        in_specs=[
            pl.BlockSpec((BN, H), lambda i: (i, 0)),
            pl.BlockSpec((BN, 16), lambda i: (i, 0)),
            pl.BlockSpec((BN, 16), lambda i: (i, 0)),
            pl.BlockSpec((BN, H), lambda i: (i, 0)),
            pl.BlockSpec((H, H), lambda i: (0, 0)),
            pl.BlockSpec((H, H), lambda i: (0, 0)),
            pl.BlockSpec((1, H), lambda i: (0, 0)),
        ],
        out_specs=[
            pl.BlockSpec((BN, H), lambda i: (i, 0)),
            pl.BlockSpec((1, 1), lambda i: (0, 0)),
            pl.BlockSpec((1, 1), lambda i: (0, 0)),
        ],
        out_shape=[
            jax.ShapeDtypeStruct((N, H), jnp.float32),
            jax.ShapeDtypeStruct((1, 1), jnp.float32),
            jax.ShapeDtypeStruct((1, 1), jnp.float32),
        ],
    )(p, d0, d1, h, wlt, wrt, b)


def _tc_norm_body(o, s, q, w, bb, h_ref):
    scale = 1.0 / (N * H)
    mu = s[...] * scale
    var = q[...] * scale - mu * mu
    inv = lax.rsqrt(var + 1e-5)
    h_ref[...] = (o[...] - mu) * inv * w[...] + bb[...]


def _tc_norm(o, s, q, lnw, lnb):
    return pl.pallas_call(
        _tc_norm_body,
        grid=(NB,),
        in_specs=[
            pl.BlockSpec((BN, H), lambda i: (i, 0)),
            pl.BlockSpec((1, 1), lambda i: (0, 0)),
            pl.BlockSpec((1, 1), lambda i: (0, 0)),
            pl.BlockSpec((1, H), lambda i: (0, 0)),
            pl.BlockSpec((1, H), lambda i: (0, 0)),
        ],
        out_specs=pl.BlockSpec((BN, H), lambda i: (i, 0)),
        out_shape=jax.ShapeDtypeStruct((N, H), jnp.float32),
    )(o, s, q, lnw, lnb)


def kernel(x, edge_index, edge_attr, W_pre, b_pre, Wl, Wr, b_conv, ln_w, ln_b):
    src = edge_index[0]
    dst = edge_index[1]
    pad = EPAD - E
    src_flat = jnp.concatenate([src, jnp.zeros((pad,), jnp.int32)])
    trash = N + jnp.mod(jnp.arange(pad, dtype=jnp.int32), NPAD - N)
    dst_flat = jnp.concatenate([dst, trash])
    dst_p32 = dst_flat.reshape(NW, CH, CHUNK)
    src_p = src_flat.reshape(16, CHA, CHUNK)
    dst_p = dst_flat.reshape(16, CHA, CHUNK)

    hidden = _tc_pre(x, W_pre.T, b_pre[None, :])

    dp = _sc_degree(dst_p32)
    d0 = dp[0, :N, :16]
    d1 = dp[1, :N, :16]

    for i in range(L):
        part = _sc_segsum(hidden, src_p, dst_p)
        o, s, q = _tc_layer(part, d0, d1, hidden,
                            Wl[i].T, Wr[i].T, b_conv[i][None, :])
        hidden = _tc_norm(o, s, q, ln_w[None, :], ln_b[None, :])
    return hidden
